# baseline (speedup 1.0000x reference)
import jax
import jax.numpy as jnp
from jax.experimental import pallas as pl
from jax.experimental.pallas import tpu as pltpu

_R = 128
_N = 4096
_C = 256
_K = 10
_S = 8
_W = _N // _S
_INF = 3.0e38
_BIGI = 1 << 30


def _loss_tile_kernel(xyz_row_ref, xyzT_ref, skel_ref, skelT_ref, out_ref):
    t = pl.program_id(1)

    rows = xyz_row_ref[0]
    px = rows[:, 0:1]
    py = rows[:, 1:2]
    pz = rows[:, 2:3]
    ax = rows[:, 3:4]
    ay = rows[:, 4:5]
    az = rows[:, 5:6]
    colsT = xyzT_ref[0]

    pn = px * px + py * py + pz * pz
    ax2 = ax * ax
    ay2 = ay * ay
    az2 = az * az
    an = ax2 + ay2 + az2

    def slab(s):
        qx = colsT[0:1, s * _W:(s + 1) * _W]
        qy = colsT[1:2, s * _W:(s + 1) * _W]
        qz = colsT[2:3, s * _W:(s + 1) * _W]
        bx = colsT[3:4, s * _W:(s + 1) * _W]
        by = colsT[4:5, s * _W:(s + 1) * _W]
        bz = colsT[5:6, s * _W:(s + 1) * _W]
        hqn = 0.5 * (qx * qx + qy * qy + qz * qz)
        bx2 = bx * bx
        by2 = by * by
        bz2 = bz * bz
        bn = bx2 + by2 + bz2

        dot = px * qx + py * qy + pz * qz
        d = hqn - dot

        g = ax * bx + ay * by + az * bz
        c2 = an * bn - g * g
        mm = ax2 * bx2 + ay2 * by2 + az2 * bz2
        v = jnp.minimum(c2, mm)
        return d, v

    def ce(d1, v1, d2, v2):
        sel = d2 < d1
        return (jnp.minimum(d1, d2), jnp.where(sel, v2, v1),
                jnp.maximum(d1, d2), jnp.where(sel, v1, v2))

    def lo(d1, v1, d2, v2):
        sel = d2 < d1
        return jnp.minimum(d1, d2), jnp.where(sel, v2, v1)

    def grp(s0):
        da, va = slab(s0)
        db, vb = slab(s0 + 1)
        a1, av1, a2, av2 = ce(da, va, db, vb)
        dc, vc = slab(s0 + 2)
        dd, vd = slab(s0 + 3)
        b1, bv1, b2, bv2 = ce(dc, vc, dd, vd)
        c, cv = lo(a2, av2, b2, bv2)
        s1, sv1, t, tv = ce(a1, av1, b1, bv1)
        s2, sv2, s3, sv3 = ce(t, tv, c, cv)
        return s1, sv1, s2, sv2, s3, sv3

    x1, xv1, x2, xv2, x3, xv3 = grp(0)
    y1, yv1, y2, yv2, y3, yv3 = grp(4)
    m1, vm1, h1, hv1 = ce(x1, xv1, y1, yv1)
    l2, lv2 = lo(x2, xv2, y2, yv2)
    l3, lv3 = lo(x3, xv3, y3, yv3)
    m2, vm2, q, qv = ce(h1, hv1, l2, lv2)
    m3, vm3 = lo(q, qv, l3, lv3)

    k1, k2, k3 = m1, m2, m3
    for _ in range(_K):
        m = jnp.min(k1, axis=1, keepdims=True)
        eq = k1 == m
        k1 = jnp.where(eq, k2, k1)
        k2 = jnp.where(eq, k3, k2)
        k3 = jnp.where(eq, _INF, k3)
    sv1 = jnp.sqrt(jnp.maximum(vm1, 1e-12))
    sv2 = jnp.sqrt(jnp.maximum(vm2, 1e-12))
    sv3 = jnp.sqrt(jnp.maximum(vm3, 1e-12))
    taken = (jnp.where(k3 == _INF, sv1, 0.0)
             + jnp.where(k2 == _INF, sv2, 0.0)
             + jnp.where(k1 == _INF, sv3, 0.0))
    acc = jnp.sum(taken, axis=1, keepdims=True)

    sT = skelT_ref[0]
    sx = sT[0:1, :]
    sy = sT[1:2, :]
    sz = sT[2:3, :]
    sn = sx * sx + sy * sy + sz * sz
    d2 = pn + sn - 2.0 * (px * sx + py * sy + pz * sz)
    iota2 = jax.lax.broadcasted_iota(jnp.int32, (_R, _C), 1)
    m1v = jnp.min(d2, axis=1, keepdims=True)
    f1 = jnp.min(jnp.where(d2 == m1v, iota2, _BIGI), axis=1, keepdims=True)
    m2v = jnp.min(jnp.where(iota2 == f1, _INF, d2), axis=1, keepdims=True)
    voro = jnp.sum(acc * (m2v - m1v))

    out_ref[0, 0, 0] = voro

    @pl.when(t == 0)
    def _spread():
        skl = skel_ref[0]
        kx = skl[:, 0:1]
        ky = skl[:, 1:2]
        kz = skl[:, 2:3]
        kn = kx * kx + ky * ky + kz * kz
        d3 = kn + sn - 2.0 * (kx * sx + ky * sy + kz * sz)
        iota3 = jax.lax.broadcasted_iota(jnp.int32, (_C, _C), 1)
        m1c = jnp.min(d3, axis=1, keepdims=True)
        f1c = jnp.min(jnp.where(d3 == m1c, iota3, _BIGI), axis=1,
                      keepdims=True)
        m2c = jnp.min(jnp.where(iota3 == f1c, _INF, d3), axis=1,
                      keepdims=True)
        chosen = jnp.sum(jnp.sqrt(jnp.maximum(m2c, 1e-12)))
        out_ref[0, 0, 0] = voro - 0.5 * chosen


def kernel(xyz, num_class, skel_xyz):
    B = xyz.shape[0]
    T = _N // _R
    xyzT = jnp.transpose(xyz, (0, 2, 1))
    skelT = jnp.transpose(skel_xyz, (0, 2, 1))

    parts = pl.pallas_call(
        _loss_tile_kernel,
        grid=(B, T),
        in_specs=[
            pl.BlockSpec((1, _R, 6), lambda b, t: (b, t, 0)),
            pl.BlockSpec((1, 6, _N), lambda b, t: (b, 0, 0)),
            pl.BlockSpec((1, _C, 3), lambda b, t: (b, 0, 0)),
            pl.BlockSpec((1, 3, _C), lambda b, t: (b, 0, 0)),
        ],
        out_specs=pl.BlockSpec((1, 1, 1), lambda b, t: (b * T + t, 0, 0),
                               memory_space=pltpu.SMEM),
        out_shape=jax.ShapeDtypeStruct((B * T, 1, 1), jnp.float32),
        compiler_params=pltpu.CompilerParams(
            dimension_semantics=("parallel", "parallel"),
        ),
    )(xyz, xyzT, skel_xyz, skelT)
    return jnp.sum(parts)


# speedup vs baseline: 1.1394x; 1.1394x over previous
import jax
import jax.numpy as jnp
from jax.experimental import pallas as pl
from jax.experimental.pallas import tpu as pltpu

_R = 256
_N = 4096
_C = 256
_K = 10
_S = 8
_W = _N // _S
_INF = 3.0e38
_BIGI = 1 << 30


def _loss_tile_kernel(xyz_row_ref, xyzT_ref, skel_ref, skelT_ref, out_ref):
    t = pl.program_id(1)

    rows = xyz_row_ref[0]
    px = rows[:, 0:1]
    py = rows[:, 1:2]
    pz = rows[:, 2:3]
    ax = rows[:, 3:4]
    ay = rows[:, 4:5]
    az = rows[:, 5:6]
    colsT = xyzT_ref[0]

    pn = px * px + py * py + pz * pz
    ax2 = ax * ax
    ay2 = ay * ay
    az2 = az * az
    an = ax2 + ay2 + az2

    def slab(s):
        qx = colsT[0:1, s * _W:(s + 1) * _W]
        qy = colsT[1:2, s * _W:(s + 1) * _W]
        qz = colsT[2:3, s * _W:(s + 1) * _W]
        bx = colsT[3:4, s * _W:(s + 1) * _W]
        by = colsT[4:5, s * _W:(s + 1) * _W]
        bz = colsT[5:6, s * _W:(s + 1) * _W]
        hqn = 0.5 * (qx * qx + qy * qy + qz * qz)
        bx2 = bx * bx
        by2 = by * by
        bz2 = bz * bz
        bn = bx2 + by2 + bz2

        dot = px * qx + py * qy + pz * qz
        d = hqn - dot

        g = ax * bx + ay * by + az * bz
        c2 = an * bn - g * g
        mm = ax2 * bx2 + ay2 * by2 + az2 * bz2
        v = jnp.minimum(c2, mm)
        return d, v

    def ce(d1, v1, d2, v2):
        sel = d2 < d1
        return (jnp.minimum(d1, d2), jnp.where(sel, v2, v1),
                jnp.maximum(d1, d2), jnp.where(sel, v1, v2))

    def lo(d1, v1, d2, v2):
        sel = d2 < d1
        return jnp.minimum(d1, d2), jnp.where(sel, v2, v1)

    def grp(s0):
        da, va = slab(s0)
        db, vb = slab(s0 + 1)
        a1, av1, a2, av2 = ce(da, va, db, vb)
        dc, vc = slab(s0 + 2)
        dd, vd = slab(s0 + 3)
        b1, bv1, b2, bv2 = ce(dc, vc, dd, vd)
        c, cv = lo(a2, av2, b2, bv2)
        s1, sv1, t, tv = ce(a1, av1, b1, bv1)
        s2, sv2, s3, sv3 = ce(t, tv, c, cv)
        return s1, sv1, s2, sv2, s3, sv3

    x1, xv1, x2, xv2, x3, xv3 = grp(0)
    y1, yv1, y2, yv2, y3, yv3 = grp(4)
    m1, vm1, h1, hv1 = ce(x1, xv1, y1, yv1)
    l2, lv2 = lo(x2, xv2, y2, yv2)
    l3, lv3 = lo(x3, xv3, y3, yv3)
    m2, vm2, q, qv = ce(h1, hv1, l2, lv2)
    m3, vm3 = lo(q, qv, l3, lv3)

    k1, k2, k3 = m1, m2, m3
    for it in range(_K):
        if it == 0:
            m = -0.5 * pn
        else:
            m = jnp.min(k1, axis=1, keepdims=True)
        eq = k1 == m
        k1 = jnp.where(eq, k2, k1)
        k2 = jnp.where(eq, k3, k2)
        k3 = jnp.where(eq, _INF, k3)
    sv1 = jnp.sqrt(jnp.maximum(vm1, 1e-12))
    sv2 = jnp.sqrt(jnp.maximum(vm2, 1e-12))
    sv3 = jnp.sqrt(jnp.maximum(vm3, 1e-12))
    taken = (jnp.where(k3 == _INF, sv1, 0.0)
             + jnp.where(k2 == _INF, sv2, 0.0)
             + jnp.where(k1 == _INF, sv3, 0.0))
    acc = jnp.sum(taken, axis=1, keepdims=True)

    sT = skelT_ref[0]
    sx = sT[0:1, :]
    sy = sT[1:2, :]
    sz = sT[2:3, :]
    sn = sx * sx + sy * sy + sz * sz
    d2 = pn + sn - 2.0 * (px * sx + py * sy + pz * sz)
    iota2 = jax.lax.broadcasted_iota(jnp.int32, (_R, _C), 1)
    m1v = jnp.min(d2, axis=1, keepdims=True)
    f1 = jnp.min(jnp.where(d2 == m1v, iota2, _BIGI), axis=1, keepdims=True)
    m2v = jnp.min(jnp.where(iota2 == f1, _INF, d2), axis=1, keepdims=True)
    voro = jnp.sum(acc * (m2v - m1v))

    out_ref[0, 0, 0] = voro

    @pl.when(t == 0)
    def _spread():
        skl = skel_ref[0]
        kx = skl[:, 0:1]
        ky = skl[:, 1:2]
        kz = skl[:, 2:3]
        kn = kx * kx + ky * ky + kz * kz
        d3 = kn + sn - 2.0 * (kx * sx + ky * sy + kz * sz)
        iota3 = jax.lax.broadcasted_iota(jnp.int32, (_C, _C), 1)
        m1c = jnp.min(d3, axis=1, keepdims=True)
        f1c = jnp.min(jnp.where(d3 == m1c, iota3, _BIGI), axis=1,
                      keepdims=True)
        m2c = jnp.min(jnp.where(iota3 == f1c, _INF, d3), axis=1,
                      keepdims=True)
        chosen = jnp.sum(jnp.sqrt(jnp.maximum(m2c, 1e-12)))
        out_ref[0, 0, 0] = voro - 0.5 * chosen


def kernel(xyz, num_class, skel_xyz):
    B = xyz.shape[0]
    T = _N // _R
    xyzT = jnp.transpose(xyz, (0, 2, 1))
    skelT = jnp.transpose(skel_xyz, (0, 2, 1))

    parts = pl.pallas_call(
        _loss_tile_kernel,
        grid=(B, T),
        in_specs=[
            pl.BlockSpec((1, _R, 6), lambda b, t: (b, t, 0)),
            pl.BlockSpec((1, 6, _N), lambda b, t: (b, 0, 0)),
            pl.BlockSpec((1, _C, 3), lambda b, t: (b, 0, 0)),
            pl.BlockSpec((1, 3, _C), lambda b, t: (b, 0, 0)),
        ],
        out_specs=pl.BlockSpec((1, 1, 1), lambda b, t: (b * T + t, 0, 0),
                               memory_space=pltpu.SMEM),
        out_shape=jax.ShapeDtypeStruct((B * T, 1, 1), jnp.float32),
        compiler_params=pltpu.CompilerParams(
            dimension_semantics=("parallel", "parallel"),
        ),
    )(xyz, xyzT, skel_xyz, skelT)
    return jnp.sum(parts)
